# baseline (speedup 1.0000x reference)
import functools

import jax
import jax.numpy as jnp
from jax import lax
from jax.experimental import pallas as pl
from jax.experimental.pallas import tpu as pltpu
from jax.experimental.pallas import tpu_sc as plsc

N = 10000
E = 320000
D = 128

NC = 2
NS = 16
NPC = N // NC
TRASH = NPC
ACC_ROWS = NPC + 8
CHUNK = 128
NCH = 160
EPW = NCH * CHUNK
EP = NS * EPW
_N_OUT_CHUNKS = -(-NPC // CHUNK)
_LAST_ROWS = NPC - (_N_OUT_CHUNKS - 1) * CHUNK
_N_OUT_STEPS = -(-_N_OUT_CHUNKS // NS)
_ACC_LAST = ACC_ROWS - (_N_OUT_CHUNKS - 1) * CHUNK


def _matmul_tc(x, w):
    bm = 1000

    def body(x_ref, w_ref, o_ref):
        o_ref[...] = jnp.dot(x_ref[...], w_ref[...],
                             preferred_element_type=jnp.float32,
                             precision=lax.Precision.HIGHEST)

    return pl.pallas_call(
        body,
        grid=(N // bm,),
        in_specs=[
            pl.BlockSpec((bm, D), lambda i: (i, 0)),
            pl.BlockSpec((D, D), lambda i: (0, 0)),
        ],
        out_specs=pl.BlockSpec((bm, D), lambda i: (i, 0)),
        out_shape=jax.ShapeDtypeStruct((N, D), jnp.float32),
    )(x, w)


def _aggregate_sc(support, packed2, wts2):
    mesh = plsc.VectorSubcoreMesh(core_axis_name="c", subcore_axis_name="s")

    @functools.partial(
        pl.kernel,
        out_type=jax.ShapeDtypeStruct((N, D), jnp.float32),
        mesh=mesh,
        scratch_types=[
            pltpu.VMEM((EPW + CHUNK,), jnp.int32),
            pltpu.VMEM((EPW + CHUNK,), jnp.float32),
            pltpu.VMEM((2, CHUNK, D), jnp.float32),
            pltpu.VMEM((1, CHUNK), jnp.int32),
            pltpu.VMEM((2, CHUNK), jnp.int32),
            pltpu.SMEM((1,), jnp.int32),
            pltpu.SemaphoreType.DMA,
            pltpu.SemaphoreType.DMA,
            pltpu.VMEM_SHARED((ACC_ROWS, D), jnp.float32),
        ],
        compiler_params=pltpu.CompilerParams(needs_layout_passes=False),
    )
    def agg(sup_hbm, pk_hbm, wts_hbm, out_hbm,
            pk_v, w_v, gbuf, idx2d, cid2d, offs, sem0, sem1, acc):
        c = lax.axis_index("c")
        s = lax.axis_index("s")
        sems = (sem0, sem1)

        pltpu.sync_copy(pk_hbm.at[s], pk_v.at[pl.ds(0, EPW)])
        pltpu.sync_copy(wts_hbm.at[s], w_v.at[pl.ds(0, EPW)])

        base = c * NPC
        offs[0] = 0

        @pl.loop(0, EPW // 16)
        def _(b):
            p = b * 16
            pk = pk_v[pl.ds(p, 16)]
            wv = w_v[pl.ds(p, 16)]
            loc = (pk >> 16) - base
            inb = (loc >= 0) & (loc < NPC)
            pk_loc = (loc << 16) | (pk & 0xFFFF)
            o = offs[0]
            plsc.store_compressed(pk_v.at[pl.ds(o, 16)], pk_loc, mask=inb)
            plsc.store_compressed(w_v.at[pl.ds(o, 16)], wv, mask=inb)
            cnt = plsc.all_reduce_population_count(inb)
            offs[0] = o + cnt[0]

        ntot = offs[0]
        trash16 = jnp.full((16,), TRASH << 16, jnp.int32)
        zeros16 = jnp.zeros((16,), jnp.float32)
        for i in range(CHUNK // 16):
            pk_v[pl.ds(ntot + 16 * i, 16)] = trash16
            w_v[pl.ds(ntot + 16 * i, 16)] = zeros16
        nchunks = (ntot + CHUNK - 1) // CHUNK

        @pl.loop(0, CHUNK)
        def _(r):
            for j in range(D // 16):
                gbuf[0, r, pl.ds(j * 16, 16)] = zeros16

        for j in range(_N_OUT_STEPS):
            k = s + NS * j

            @pl.when(k < _N_OUT_CHUNKS - 1)
            def _():
                pltpu.sync_copy(gbuf.at[0], acc.at[pl.ds(k * CHUNK, CHUNK)])

            @pl.when(k == _N_OUT_CHUNKS - 1)
            def _():
                pltpu.sync_copy(gbuf.at[0, pl.ds(0, _ACC_LAST)],
                                acc.at[pl.ds(k * CHUNK, _ACC_LAST)])
        plsc.subcore_barrier()

        def prep_and_fire(chn, b):
            for j in range(CHUNK // 16):
                cid2d[b, pl.ds(j * 16, 16)] = (
                    pk_v[pl.ds(chn * CHUNK + j * 16, 16)] & 0xFFFF)
            pltpu.async_copy(sup_hbm.at[cid2d.at[b]], gbuf.at[b], sems[b])

        def process(chn, b):
            e_base = chn * CHUNK
            pltpu.make_async_copy(
                sup_hbm.at[pl.ds(0, CHUNK)], gbuf.at[b], sems[b]).wait()

            @pl.loop(0, CHUNK, step=16)
            def _(e0):
                wv = w_v[pl.ds(e_base + e0, 16)]
                for i in range(16):
                    w = wv[i]
                    for j in range(D // 16):
                        sl = pl.ds(j * 16, 16)
                        gbuf[b, e0 + i, sl] = gbuf[b, e0 + i, sl] * w

            for j in range(CHUNK // 16):
                idx2d[0, pl.ds(j * 16, 16)] = (
                    pk_v[pl.ds(e_base + j * 16, 16)] >> 16)

            pltpu.sync_copy(gbuf.at[b], acc.at[idx2d.at[0]], add=True)

        @pl.when(nchunks > 0)
        def _():
            prep_and_fire(0, 0)

        @pl.loop(0, NCH, step=2)
        def _(ch0):
            for b in range(2):
                chn = ch0 + b

                @pl.when(chn < nchunks)
                def _(chn=chn, b=b):
                    @pl.when(chn + 1 < nchunks)
                    def _():
                        prep_and_fire(chn + 1, 1 - b)

                    process(chn, b)

        plsc.subcore_barrier()

        for j in range(_N_OUT_STEPS):
            k = s + NS * j

            @pl.when(k < _N_OUT_CHUNKS - 1)
            def _():
                pltpu.sync_copy(acc.at[pl.ds(k * CHUNK, CHUNK)],
                                out_hbm.at[pl.ds(base + k * CHUNK, CHUNK)])

            @pl.when(k == _N_OUT_CHUNKS - 1)
            def _():
                pltpu.sync_copy(acc.at[pl.ds(k * CHUNK, _LAST_ROWS)],
                                out_hbm.at[pl.ds(base + k * CHUNK, _LAST_ROWS)])

    return agg(support, packed2, wts2)


def _finish_tc(agg_out, alpha):
    bm = 1000

    def body(p_ref, a_ref, o_ref):
        t = p_ref[...]
        a = a_ref[0, 0]
        o_ref[...] = jnp.where(t >= 0, t, a * t)

    return pl.pallas_call(
        body,
        grid=(N // bm,),
        in_specs=[
            pl.BlockSpec((bm, D), lambda i: (i, 0)),
            pl.BlockSpec(memory_space=pltpu.SMEM),
        ],
        out_specs=pl.BlockSpec((bm, D), lambda i: (i, 0)),
        out_shape=jax.ShapeDtypeStruct((N, D), jnp.float32),
    )(agg_out, alpha.reshape(1, 1))


def kernel(node_x, edge_index, edge_weight, W, alpha):
    support = _matmul_tc(node_x, W)

    pad = EP - E
    packed = (edge_index[0] << 16) | edge_index[1]
    packed_p = jnp.concatenate(
        [packed, jnp.full((pad,), N << 16, jnp.int32)])
    w_p = jnp.concatenate([edge_weight, jnp.zeros((pad,), jnp.float32)])
    packed2 = packed_p.reshape(NS, EPW)
    wts2 = w_p.reshape(NS, EPW)

    agg_out = _aggregate_sc(support, packed2, wts2)
    act = _finish_tc(agg_out, alpha)
    return act, support


# speedup vs baseline: 1.0201x; 1.0201x over previous
import functools

import jax
import jax.numpy as jnp
from jax import lax
from jax.experimental import pallas as pl
from jax.experimental.pallas import tpu as pltpu
from jax.experimental.pallas import tpu_sc as plsc

N = 10000
E = 320000
D = 128

NC = 2
NS = 16
NPC = N // NC
TRASH = NPC
ACC_ROWS = NPC + 8
CHUNK = 128
NCH = 160
EPW = NCH * CHUNK
EP = NS * EPW
_N_OUT_CHUNKS = -(-NPC // CHUNK)
_LAST_ROWS = NPC - (_N_OUT_CHUNKS - 1) * CHUNK
_N_OUT_STEPS = -(-_N_OUT_CHUNKS // NS)
_ACC_LAST = ACC_ROWS - (_N_OUT_CHUNKS - 1) * CHUNK


def _matmul_tc(x, w):
    bm = 1000

    def body(x_ref, w_ref, o_ref):
        o_ref[...] = jnp.dot(x_ref[...], w_ref[...],
                             preferred_element_type=jnp.float32,
                             precision=lax.Precision.HIGHEST)

    return pl.pallas_call(
        body,
        grid=(N // bm,),
        in_specs=[
            pl.BlockSpec((bm, D), lambda i: (i, 0)),
            pl.BlockSpec((D, D), lambda i: (0, 0)),
        ],
        out_specs=pl.BlockSpec((bm, D), lambda i: (i, 0)),
        out_shape=jax.ShapeDtypeStruct((N, D), jnp.float32),
    )(x, w)


def _aggregate_sc(support, packed2, wts2, alpha16):
    mesh = plsc.VectorSubcoreMesh(core_axis_name="c", subcore_axis_name="s")

    @functools.partial(
        pl.kernel,
        out_type=jax.ShapeDtypeStruct((N, D), jnp.float32),
        mesh=mesh,
        scratch_types=[
            pltpu.VMEM((EPW + CHUNK,), jnp.int32),
            pltpu.VMEM((EPW + CHUNK,), jnp.float32),
            pltpu.VMEM((2, CHUNK, D), jnp.float32),
            pltpu.VMEM((1, CHUNK), jnp.int32),
            pltpu.VMEM((2, CHUNK), jnp.int32),
            pltpu.SMEM((1,), jnp.int32),
            pltpu.SemaphoreType.DMA,
            pltpu.SemaphoreType.DMA,
            pltpu.VMEM((16,), jnp.float32),
            pltpu.VMEM_SHARED((ACC_ROWS, D), jnp.float32),
        ],
        compiler_params=pltpu.CompilerParams(needs_layout_passes=False),
    )
    def agg(sup_hbm, pk_hbm, wts_hbm, alpha_hbm, out_hbm,
            pk_v, w_v, gbuf, idx2d, cid2d, offs, sem0, sem1, alpha_v, acc):
        c = lax.axis_index("c")
        s = lax.axis_index("s")
        sems = (sem0, sem1)

        pltpu.async_copy(pk_hbm.at[s], pk_v.at[pl.ds(0, EPW)], sem0)
        pltpu.async_copy(wts_hbm.at[s], w_v.at[pl.ds(0, EPW)], sem1)
        pltpu.sync_copy(alpha_hbm, alpha_v)

        zeros16 = jnp.zeros((16,), jnp.float32)

        @pl.loop(0, CHUNK)
        def _(r):
            for j in range(D // 16):
                gbuf[0, r, pl.ds(j * 16, 16)] = zeros16

        for j in range(_N_OUT_STEPS):
            k = s + NS * j

            @pl.when(k < _N_OUT_CHUNKS - 1)
            def _():
                pltpu.sync_copy(gbuf.at[0], acc.at[pl.ds(k * CHUNK, CHUNK)])

            @pl.when(k == _N_OUT_CHUNKS - 1)
            def _():
                pltpu.sync_copy(gbuf.at[0, pl.ds(0, _ACC_LAST)],
                                acc.at[pl.ds(k * CHUNK, _ACC_LAST)])

        pltpu.make_async_copy(pk_hbm.at[s], pk_v.at[pl.ds(0, EPW)], sem0).wait()
        pltpu.make_async_copy(wts_hbm.at[s], w_v.at[pl.ds(0, EPW)], sem1).wait()

        base = c * NPC
        offs[0] = 0

        @pl.loop(0, EPW // 16)
        def _(b):
            p = b * 16
            pk = pk_v[pl.ds(p, 16)]
            wv = w_v[pl.ds(p, 16)]
            loc = (pk >> 16) - base
            inb = (loc >= 0) & (loc < NPC)
            pk_loc = (loc << 16) | (pk & 0xFFFF)
            o = offs[0]
            plsc.store_compressed(pk_v.at[pl.ds(o, 16)], pk_loc, mask=inb)
            plsc.store_compressed(w_v.at[pl.ds(o, 16)], wv, mask=inb)
            cnt = plsc.all_reduce_population_count(inb)
            offs[0] = o + cnt[0]

        ntot = offs[0]
        trash16 = jnp.full((16,), TRASH << 16, jnp.int32)
        for i in range(CHUNK // 16):
            pk_v[pl.ds(ntot + 16 * i, 16)] = trash16
            w_v[pl.ds(ntot + 16 * i, 16)] = zeros16
        nchunks = (ntot + CHUNK - 1) // CHUNK
        plsc.subcore_barrier()

        def prep_and_fire(chn, b):
            for j in range(CHUNK // 16):
                cid2d[b, pl.ds(j * 16, 16)] = (
                    pk_v[pl.ds(chn * CHUNK + j * 16, 16)] & 0xFFFF)
            pltpu.async_copy(sup_hbm.at[cid2d.at[b]], gbuf.at[b], sems[b])

        def process(chn, b):
            e_base = chn * CHUNK
            pltpu.make_async_copy(
                sup_hbm.at[pl.ds(0, CHUNK)], gbuf.at[b], sems[b]).wait()

            @pl.loop(0, CHUNK, step=16)
            def _(e0):
                wv = w_v[pl.ds(e_base + e0, 16)]
                for i in range(16):
                    w = wv[i]
                    for j in range(D // 16):
                        sl = pl.ds(j * 16, 16)
                        gbuf[b, e0 + i, sl] = gbuf[b, e0 + i, sl] * w

            for j in range(CHUNK // 16):
                idx2d[0, pl.ds(j * 16, 16)] = (
                    pk_v[pl.ds(e_base + j * 16, 16)] >> 16)

            pltpu.sync_copy(gbuf.at[b], acc.at[idx2d.at[0]], add=True)

        @pl.when(nchunks > 0)
        def _():
            prep_and_fire(0, 0)

        @pl.loop(0, NCH, step=2)
        def _(ch0):
            for b in range(2):
                chn = ch0 + b

                @pl.when(chn < nchunks)
                def _(chn=chn, b=b):
                    @pl.when(chn + 1 < nchunks)
                    def _():
                        prep_and_fire(chn + 1, 1 - b)

                    process(chn, b)

        plsc.subcore_barrier()

        av = alpha_v[...]

        def prelu_rows(nrows):
            @pl.loop(0, nrows)
            def _(r):
                for j in range(D // 16):
                    sl = pl.ds(j * 16, 16)
                    t = gbuf[0, r, sl]
                    gbuf[0, r, sl] = jnp.where(t >= 0, t, av * t)

        for j in range(_N_OUT_STEPS):
            k = s + NS * j

            @pl.when(k < _N_OUT_CHUNKS - 1)
            def _():
                pltpu.sync_copy(acc.at[pl.ds(k * CHUNK, CHUNK)], gbuf.at[0])
                prelu_rows(CHUNK)
                pltpu.sync_copy(gbuf.at[0],
                                out_hbm.at[pl.ds(base + k * CHUNK, CHUNK)])

            @pl.when(k == _N_OUT_CHUNKS - 1)
            def _():
                pltpu.sync_copy(acc.at[pl.ds(k * CHUNK, _LAST_ROWS)],
                                gbuf.at[0, pl.ds(0, _LAST_ROWS)])
                prelu_rows(_LAST_ROWS)
                pltpu.sync_copy(gbuf.at[0, pl.ds(0, _LAST_ROWS)],
                                out_hbm.at[pl.ds(base + k * CHUNK, _LAST_ROWS)])

    return agg(support, packed2, wts2, alpha16)


def kernel(node_x, edge_index, edge_weight, W, alpha):
    support = _matmul_tc(node_x, W)

    pad = EP - E
    packed = (edge_index[0] << 16) | edge_index[1]
    packed_p = jnp.concatenate(
        [packed, jnp.full((pad,), N << 16, jnp.int32)])
    w_p = jnp.concatenate([edge_weight, jnp.zeros((pad,), jnp.float32)])
    packed2 = packed_p.reshape(NS, EPW)
    wts2 = w_p.reshape(NS, EPW)

    alpha16 = jnp.full((16,), alpha, jnp.float32)
    act = _aggregate_sc(support, packed2, wts2, alpha16)
    return act, support


# speedup vs baseline: 1.0444x; 1.0238x over previous
import functools

import jax
import jax.numpy as jnp
from jax import lax
from jax.experimental import pallas as pl
from jax.experimental.pallas import tpu as pltpu
from jax.experimental.pallas import tpu_sc as plsc

N = 10000
E = 320000
D = 128

NC = 2
NS = 16
NPC = N // NC
TRASH = NPC
ACC_ROWS = NPC + 8
CHUNK = 128
NCH = 160
EPW = NCH * CHUNK
EP = NS * EPW
_N_OUT_CHUNKS = -(-NPC // CHUNK)
_LAST_ROWS = NPC - (_N_OUT_CHUNKS - 1) * CHUNK
_N_OUT_STEPS = -(-_N_OUT_CHUNKS // NS)
_ACC_LAST = ACC_ROWS - (_N_OUT_CHUNKS - 1) * CHUNK


def _matmul_tc(x, w):
    bm = 1000

    def body(x_ref, w_ref, o_ref):
        o_ref[...] = jnp.dot(x_ref[...], w_ref[...],
                             preferred_element_type=jnp.float32,
                             precision=lax.Precision.HIGHEST)

    return pl.pallas_call(
        body,
        grid=(N // bm,),
        in_specs=[
            pl.BlockSpec((bm, D), lambda i: (i, 0)),
            pl.BlockSpec((D, D), lambda i: (0, 0)),
        ],
        out_specs=pl.BlockSpec((bm, D), lambda i: (i, 0)),
        out_shape=jax.ShapeDtypeStruct((N, D), jnp.float32),
    )(x, w)


def _aggregate_sc(support, packed2, wts2, alpha16):
    mesh = plsc.VectorSubcoreMesh(core_axis_name="c", subcore_axis_name="s")

    @functools.partial(
        pl.kernel,
        out_type=jax.ShapeDtypeStruct((N, D), jnp.float32),
        mesh=mesh,
        scratch_types=[
            pltpu.VMEM((EPW + CHUNK,), jnp.int32),
            pltpu.VMEM((EPW + CHUNK,), jnp.float32),
            pltpu.VMEM((3, CHUNK, D), jnp.float32),
            pltpu.VMEM((3, CHUNK), jnp.int32),
            pltpu.SMEM((1,), jnp.int32),
            pltpu.SemaphoreType.DMA,
            pltpu.SemaphoreType.DMA,
            pltpu.SemaphoreType.DMA,
            pltpu.VMEM((16,), jnp.float32),
            pltpu.VMEM_SHARED((ACC_ROWS, D), jnp.float32),
        ],
        compiler_params=pltpu.CompilerParams(needs_layout_passes=False),
    )
    def agg(sup_hbm, pk_hbm, wts_hbm, alpha_hbm, out_hbm,
            pk_v, w_v, gbuf, cid2d, offs, sem0, sem1, sem2, alpha_v, acc):
        c = lax.axis_index("c")
        s = lax.axis_index("s")
        sems = (sem0, sem1, sem2)

        pltpu.async_copy(pk_hbm.at[s], pk_v.at[pl.ds(0, EPW)], sem0)
        pltpu.async_copy(wts_hbm.at[s], w_v.at[pl.ds(0, EPW)], sem1)
        pltpu.sync_copy(alpha_hbm, alpha_v)

        zeros16 = jnp.zeros((16,), jnp.float32)

        @pl.loop(0, CHUNK)
        def _(r):
            for j in range(D // 16):
                gbuf[0, r, pl.ds(j * 16, 16)] = zeros16

        for j in range(_N_OUT_STEPS):
            k = s + NS * j

            @pl.when(k < _N_OUT_CHUNKS - 1)
            def _():
                pltpu.sync_copy(gbuf.at[0], acc.at[pl.ds(k * CHUNK, CHUNK)])

            @pl.when(k == _N_OUT_CHUNKS - 1)
            def _():
                pltpu.sync_copy(gbuf.at[0, pl.ds(0, _ACC_LAST)],
                                acc.at[pl.ds(k * CHUNK, _ACC_LAST)])

        pltpu.make_async_copy(pk_hbm.at[s], pk_v.at[pl.ds(0, EPW)], sem0).wait()
        pltpu.make_async_copy(wts_hbm.at[s], w_v.at[pl.ds(0, EPW)], sem1).wait()

        base = c * NPC
        offs[0] = 0

        @pl.loop(0, EPW // 16)
        def _(b):
            p = b * 16
            pk = pk_v[pl.ds(p, 16)]
            wv = w_v[pl.ds(p, 16)]
            loc = (pk >> 16) - base
            inb = (loc >= 0) & (loc < NPC)
            pk_loc = (loc << 16) | (pk & 0xFFFF)
            o = offs[0]
            plsc.store_compressed(pk_v.at[pl.ds(o, 16)], pk_loc, mask=inb)
            plsc.store_compressed(w_v.at[pl.ds(o, 16)], wv, mask=inb)
            cnt = plsc.all_reduce_population_count(inb)
            offs[0] = o + cnt[0]

        ntot = offs[0]
        trash16 = jnp.full((16,), TRASH << 16, jnp.int32)
        for i in range(CHUNK // 16):
            pk_v[pl.ds(ntot + 16 * i, 16)] = trash16
            w_v[pl.ds(ntot + 16 * i, 16)] = zeros16
        nchunks = (ntot + CHUNK - 1) // CHUNK
        plsc.subcore_barrier()

        def prep_and_fire(chn, b):
            for j in range(CHUNK // 16):
                cid2d[b, pl.ds(j * 16, 16)] = (
                    pk_v[pl.ds(chn * CHUNK + j * 16, 16)] & 0xFFFF)
            pltpu.async_copy(sup_hbm.at[cid2d.at[b]], gbuf.at[b], sems[b])

        def process(chn, b):
            e_base = chn * CHUNK
            pltpu.make_async_copy(
                sup_hbm.at[pl.ds(0, CHUNK)], gbuf.at[b], sems[b]).wait()

            @pl.loop(0, CHUNK, step=16)
            def _(e0):
                wv = w_v[pl.ds(e_base + e0, 16)]
                for i in range(16):
                    w = wv[i]
                    for j in range(D // 16):
                        sl = pl.ds(j * 16, 16)
                        gbuf[b, e0 + i, sl] = gbuf[b, e0 + i, sl] * w

            for j in range(CHUNK // 16):
                cid2d[b, pl.ds(j * 16, 16)] = (
                    pk_v[pl.ds(e_base + j * 16, 16)] >> 16)

            pltpu.sync_copy(gbuf.at[b], acc.at[cid2d.at[b]], add=True)

        @pl.when(nchunks > 0)
        def _():
            prep_and_fire(0, 0)

        @pl.when(nchunks > 1)
        def _():
            prep_and_fire(1, 1)

        @pl.loop(0, NCH + 2, step=3)
        def _(ch0):
            for b in range(3):
                chn = ch0 + b

                @pl.when(chn < nchunks)
                def _(chn=chn, b=b):
                    @pl.when(chn + 2 < nchunks)
                    def _():
                        prep_and_fire(chn + 2, (b + 2) % 3)

                    process(chn, b)

        plsc.subcore_barrier()

        av = alpha_v[...]

        def prelu_rows(nrows):
            @pl.loop(0, nrows)
            def _(r):
                for j in range(D // 16):
                    sl = pl.ds(j * 16, 16)
                    t = gbuf[0, r, sl]
                    gbuf[0, r, sl] = jnp.where(t >= 0, t, av * t)

        for j in range(_N_OUT_STEPS):
            k = s + NS * j

            @pl.when(k < _N_OUT_CHUNKS - 1)
            def _():
                pltpu.sync_copy(acc.at[pl.ds(k * CHUNK, CHUNK)], gbuf.at[0])
                prelu_rows(CHUNK)
                pltpu.sync_copy(gbuf.at[0],
                                out_hbm.at[pl.ds(base + k * CHUNK, CHUNK)])

            @pl.when(k == _N_OUT_CHUNKS - 1)
            def _():
                pltpu.sync_copy(acc.at[pl.ds(k * CHUNK, _LAST_ROWS)],
                                gbuf.at[0, pl.ds(0, _LAST_ROWS)])
                prelu_rows(_LAST_ROWS)
                pltpu.sync_copy(gbuf.at[0, pl.ds(0, _LAST_ROWS)],
                                out_hbm.at[pl.ds(base + k * CHUNK, _LAST_ROWS)])

    return agg(support, packed2, wts2, alpha16)


def kernel(node_x, edge_index, edge_weight, W, alpha):
    support = _matmul_tc(node_x, W)

    pad = EP - E
    packed = (edge_index[0] << 16) | edge_index[1]
    packed_p = jnp.concatenate(
        [packed, jnp.full((pad,), N << 16, jnp.int32)])
    w_p = jnp.concatenate([edge_weight, jnp.zeros((pad,), jnp.float32)])
    packed2 = packed_p.reshape(NS, EPW)
    wts2 = w_p.reshape(NS, EPW)

    alpha16 = jnp.full((16,), alpha, jnp.float32)
    act = _aggregate_sc(support, packed2, wts2, alpha16)
    return act, support
